# baseline (speedup 1.0000x reference)
import jax
import jax.numpy as jnp
from jax.experimental import pallas as pl
from jax.experimental.pallas import tpu as pltpu

_THRES = 0.5
_L = 4096
_BR = 512
_BW = 512
_NEG = float("-inf")


def _reduce_body(x_ref, vals_ref, inds_ref):
    i = pl.program_id(0)
    r0 = i * _BR
    x = x_ref[...]
    rows = jax.lax.broadcasted_iota(jnp.int32, (_BR, _L), 0) + r0
    cols = jax.lax.broadcasted_iota(jnp.int32, (_BR, _L), 1)
    xm = jnp.where(rows == cols, _NEG, x)
    vals = jnp.max(xm, axis=1)
    ismax = xm == vals[:, None]
    inds = jnp.min(jnp.where(ismax, cols, _L), axis=1)
    vals_ref[0, 0, :] = vals
    inds_ref[0, 0, :] = inds


def _assemble_body(indsr_ref, valsr_ref, indsc_ref, valsc_ref, out_ref):
    i = pl.program_id(0)
    r0 = i * _BW
    inds_row = indsr_ref[...]
    vals_row = valsr_ref[...]
    inds_col = indsc_ref[...]
    vals_col = valsc_ref[...]
    rows = jax.lax.broadcasted_iota(jnp.int32, (_BW, _L), 0) + r0
    cols = jax.lax.broadcasted_iota(jnp.int32, (_BW, _L), 1)
    t1 = rows == cols
    t2 = (inds_col == cols) & (vals_col > _THRES)
    t3 = (inds_row == rows) & (vals_row > _THRES)
    out_ref[...] = jnp.where(t1 | t2 | t3, jnp.float32(1.0), jnp.float32(0.0))


def kernel(input):
    x2d = input.reshape(_L, _L)
    g1 = _L // _BR
    vals3, inds3 = pl.pallas_call(
        _reduce_body,
        grid=(g1,),
        in_specs=[pl.BlockSpec((_BR, _L), lambda i: (i, 0))],
        out_specs=[
            pl.BlockSpec((1, 1, _BR), lambda i: (i, 0, 0)),
            pl.BlockSpec((1, 1, _BR), lambda i: (i, 0, 0)),
        ],
        out_shape=[
            jax.ShapeDtypeStruct((g1, 1, _BR), jnp.float32),
            jax.ShapeDtypeStruct((g1, 1, _BR), jnp.int32),
        ],
    )(x2d)
    vals_row = vals3.reshape(1, _L)
    inds_row = inds3.reshape(1, _L)
    vals_col = vals3.reshape(_L, 1)
    inds_col = inds3.reshape(_L, 1)

    g2 = _L // _BW
    out2d = pl.pallas_call(
        _assemble_body,
        grid=(g2,),
        in_specs=[
            pl.BlockSpec((1, _L), lambda i: (0, 0)),
            pl.BlockSpec((1, _L), lambda i: (0, 0)),
            pl.BlockSpec((_BW, 1), lambda i: (i, 0)),
            pl.BlockSpec((_BW, 1), lambda i: (i, 0)),
        ],
        out_specs=pl.BlockSpec((_BW, _L), lambda i: (i, 0)),
        out_shape=jax.ShapeDtypeStruct((_L, _L), jnp.float32),
    )(inds_row, vals_row, inds_col, vals_col)
    return out2d.reshape(input.shape)


# speedup vs baseline: 1.3894x; 1.3894x over previous
import jax
import jax.numpy as jnp
from jax.experimental import pallas as pl
from jax.experimental.pallas import tpu as pltpu

_THRES = 0.5
_L = 4096
_BR = 512
_BW = 1024
_NEG = float("-inf")


def _reduce_body(x_ref, c2_ref, d2_ref):
    i = pl.program_id(0)
    r0 = i * _BR
    x = x_ref[...]
    rows = jax.lax.broadcasted_iota(jnp.int32, (_BR, _L), 0) + r0
    cols = jax.lax.broadcasted_iota(jnp.int32, (_BR, _L), 1)
    xm = jnp.where(rows == cols, _NEG, x)
    vals = jnp.max(xm, axis=1)
    ismax = xm == vals[:, None]
    inds = jnp.min(jnp.where(ismax, cols, _L), axis=1)
    msk = vals > _THRES
    rowid = jax.lax.iota(jnp.int32, _BR) + r0
    c2_ref[0, 0, :] = jnp.where(msk, inds, rowid)
    d2_ref[0, 0, :] = jnp.where(msk, inds, -1)


def _assemble_body(d2r_ref, c2c_ref, out_ref):
    i = pl.program_id(0)
    r0 = i * _BW
    d2_row = d2r_ref[...]
    c2_col = c2c_ref[...]
    rows = jax.lax.broadcasted_iota(jnp.int32, (_BW, _L), 0) + r0
    cols = jax.lax.broadcasted_iota(jnp.int32, (_BW, _L), 1)
    hit = (rows == cols) | (c2_col == cols) | (d2_row == rows)
    out_ref[...] = jnp.where(hit, jnp.float32(1.0), jnp.float32(0.0))


def kernel(input):
    x2d = input.reshape(_L, _L)
    g1 = _L // _BR
    c2_3, d2_3 = pl.pallas_call(
        _reduce_body,
        grid=(g1,),
        in_specs=[pl.BlockSpec((_BR, _L), lambda i: (i, 0))],
        out_specs=[
            pl.BlockSpec((1, 1, _BR), lambda i: (i, 0, 0)),
            pl.BlockSpec((1, 1, _BR), lambda i: (i, 0, 0)),
        ],
        out_shape=[
            jax.ShapeDtypeStruct((g1, 1, _BR), jnp.int32),
            jax.ShapeDtypeStruct((g1, 1, _BR), jnp.int32),
        ],
    )(x2d)
    d2_row = d2_3.reshape(1, _L)
    c2_col = c2_3.reshape(_L, 1)

    g2 = _L // _BW
    out2d = pl.pallas_call(
        _assemble_body,
        grid=(g2,),
        in_specs=[
            pl.BlockSpec((1, _L), lambda i: (0, 0)),
            pl.BlockSpec((_BW, 1), lambda i: (i, 0)),
        ],
        out_specs=pl.BlockSpec((_BW, _L), lambda i: (i, 0)),
        out_shape=jax.ShapeDtypeStruct((_L, _L), jnp.float32),
    )(d2_row, c2_col)
    return out2d.reshape(input.shape)


# speedup vs baseline: 1.4746x; 1.0613x over previous
import jax
import jax.numpy as jnp
from jax.experimental import pallas as pl
from jax.experimental.pallas import tpu as pltpu

_THRES = 0.5
_L = 4096
_BR = 512
_BW = 1024
_NEG = float("-inf")


def _reduce_body(x_ref, c2_ref, d2_ref):
    i = pl.program_id(0)
    r0 = i * _BR
    x = x_ref[...]
    rows = jax.lax.broadcasted_iota(jnp.int32, (_BR, _L), 0) + r0
    cols = jax.lax.broadcasted_iota(jnp.int32, (_BR, _L), 1)
    xm = jnp.where(rows == cols, _NEG, x)
    vals = jnp.max(xm, axis=1)
    ismax = xm == vals[:, None]
    inds = jnp.min(jnp.where(ismax, cols, _L), axis=1)
    msk = vals > _THRES
    rowid = jax.lax.iota(jnp.int32, _BR) + r0
    c2 = jnp.where(msk, inds, rowid)
    d2 = jnp.where(msk, inds, -1)
    c2_ref[...] = c2[:, None]
    d2_ref[...] = d2[None, :]


def _assemble_body(d2r_ref, c2c_ref, out_ref):
    i = pl.program_id(0)
    r0 = i * _BW
    d2_row = d2r_ref[...]
    c2_col = c2c_ref[...]
    rows = jax.lax.broadcasted_iota(jnp.int32, (_BW, _L), 0) + r0
    cols = jax.lax.broadcasted_iota(jnp.int32, (_BW, _L), 1)
    hit = (rows == cols) | (c2_col == cols) | (d2_row == rows)
    out_ref[...] = jnp.where(hit, jnp.float32(1.0), jnp.float32(0.0))


def kernel(input):
    x2d = input.reshape(_L, _L)
    g1 = _L // _BR
    c2_col, d2_row = pl.pallas_call(
        _reduce_body,
        grid=(g1,),
        in_specs=[pl.BlockSpec((_BR, _L), lambda i: (i, 0))],
        out_specs=[
            pl.BlockSpec((_BR, 1), lambda i: (i, 0)),
            pl.BlockSpec((1, _BR), lambda i: (0, i)),
        ],
        out_shape=[
            jax.ShapeDtypeStruct((_L, 1), jnp.int32),
            jax.ShapeDtypeStruct((1, _L), jnp.int32),
        ],
    )(x2d)

    g2 = _L // _BW
    out2d = pl.pallas_call(
        _assemble_body,
        grid=(g2,),
        in_specs=[
            pl.BlockSpec((1, _L), lambda i: (0, 0)),
            pl.BlockSpec((_BW, 1), lambda i: (i, 0)),
        ],
        out_specs=pl.BlockSpec((_BW, _L), lambda i: (i, 0)),
        out_shape=jax.ShapeDtypeStruct((_L, _L), jnp.float32),
    )(d2_row, c2_col)
    return out2d.reshape(input.shape)
